# Initial kernel scaffold; baseline (speedup 1.0000x reference)
import jax
import jax.numpy as jnp
from jax.experimental import pallas as pl


def kernel(x, time_of_day, day_of_week):
    raise NotImplementedError("write your pallas kernel here")


# speedup vs baseline: 2.5603x; 2.5603x over previous
import jax
import jax.numpy as jnp
from jax import lax
from jax.experimental import pallas as pl
from jax.experimental.pallas import tpu as pltpu
from jax.experimental.pallas import tpu_sc as plsc

B, T, N, C = 64, 12, 10000, 3
TIMES = 288
DAYS = 7
F = 64
NC, NS, L = 2, 16, 16
NW = NC * NS
B_PER_W = B // NW
NSTEP = N // L


def _body(x_hbm, tod_hbm, dow_hbm, out_hbm,
          slab_v, kidx_v, didx_v, tod_v, dow_v, row_v):
    wid = lax.axis_index("s") * NC + lax.axis_index("c")
    pltpu.sync_copy(tod_hbm, tod_v)
    pltpu.sync_copy(dow_hbm, dow_v)
    for rb in range(B_PER_W):
        b = wid * B_PER_W + rb
        pltpu.sync_copy(x_hbm.at[b, T - 1], slab_v)

        def idx_body(i, _):
            n0 = i * L
            base = 3 * n0 + 3 * lax.broadcasted_iota(jnp.int32, (L,), 0)
            v1 = plsc.load_gather(slab_v, [base + 1])
            v2 = plsc.load_gather(slab_v, [base + 2])
            kidx_v[pl.ds(n0, L)] = (v1 * TIMES).astype(jnp.int32) * F
            didx_v[pl.ds(n0, L)] = v2.astype(jnp.int32) * F
            return 0

        lax.fori_loop(0, NSTEP, idx_body, 0)

        def f_body(f, _):
            def n_body(i, _):
                n0 = i * L
                k = kidx_v[pl.ds(n0, L)]
                d = didx_v[pl.ds(n0, L)]
                row = (plsc.load_gather(tod_v, [k + f])
                       + plsc.load_gather(dow_v, [d + f]))
                row_v[pl.ds(n0, L)] = row
                return 0

            lax.fori_loop(0, NSTEP, n_body, 0)
            pltpu.sync_copy(row_v, out_hbm.at[b, f])
            return 0

        lax.fori_loop(0, F, f_body, 0)


def kernel(x, time_of_day, day_of_week):
    mesh = plsc.VectorSubcoreMesh(core_axis_name="c", subcore_axis_name="s",
                                  num_cores=NC, num_subcores=NS)
    out = pl.kernel(
        _body,
        out_type=jax.ShapeDtypeStruct((B, F, N), jnp.float32),
        mesh=mesh,
        compiler_params=pltpu.CompilerParams(needs_layout_passes=False),
        scratch_types=[
            pltpu.VMEM((N * C,), jnp.float32),
            pltpu.VMEM((N,), jnp.int32),
            pltpu.VMEM((N,), jnp.int32),
            pltpu.VMEM((TIMES * F,), jnp.float32),
            pltpu.VMEM((DAYS * F,), jnp.float32),
            pltpu.VMEM((N,), jnp.float32),
        ],
    )(x.reshape(B, T, N * C), time_of_day.reshape(-1), day_of_week.reshape(-1))
    return out[..., None]


# speedup vs baseline: 3.2908x; 1.2853x over previous
import jax
import jax.numpy as jnp
from jax import lax
from jax.experimental import pallas as pl
from jax.experimental.pallas import tpu as pltpu
from jax.experimental.pallas import tpu_sc as plsc

B, T, N, C = 64, 12, 10000, 3
TIMES = 288
DAYS = 7
F = 64
NC, NS, L = 2, 16, 16
NW = NC * NS
B_PER_W = B // NW
FQ = 4


def _body(x_hbm, tod_hbm, dow_hbm, out_hbm,
          slab_v, pidx_v, tod_v, dow_v, rows_v, sems):
    wid = lax.axis_index("s") * NC + lax.axis_index("c")
    pltpu.sync_copy(tod_hbm, tod_v)
    pltpu.sync_copy(dow_hbm, dow_v)
    for rb in range(B_PER_W):
        b = wid * B_PER_W + rb
        pltpu.sync_copy(x_hbm.at[b, T - 1], slab_v)

        @plsc.parallel_loop(0, N, step=L, unroll=5)
        def _(n0):
            base = 3 * n0 + 3 * lax.broadcasted_iota(jnp.int32, (L,), 0)
            v1 = plsc.load_gather(slab_v, [base + 1])
            v2 = plsc.load_gather(slab_v, [base + 2])
            k64 = (v1 * TIMES).astype(jnp.int32) * F
            d64 = v2.astype(jnp.int32) * F
            pidx_v[pl.ds(n0, L)] = k64 | (d64 << 16)

        def quad_body(cc, _):
            f0 = FQ * cc

            @pl.when(cc > 0)
            def _():
                for j in range(FQ):
                    pltpu.make_async_copy(
                        rows_v[j], out_hbm.at[b, 0], sems[j]).wait()

            @plsc.parallel_loop(0, N, step=L, unroll=5)
            def _(n0):
                p = pidx_v[pl.ds(n0, L)]
                k = p & 0xFFFF
                d = p >> 16
                for j in range(FQ):
                    rows_v[j][pl.ds(n0, L)] = (
                        plsc.load_gather(tod_v, [k + (f0 + j)])
                        + plsc.load_gather(dow_v, [d + (f0 + j)]))

            for j in range(FQ):
                pltpu.async_copy(rows_v[j], out_hbm.at[b, f0 + j], sems[j])
            return 0

        lax.fori_loop(0, F // FQ, quad_body, 0)
        for j in range(FQ):
            pltpu.make_async_copy(rows_v[j], out_hbm.at[b, 0], sems[j]).wait()


def kernel(x, time_of_day, day_of_week):
    mesh = plsc.VectorSubcoreMesh(core_axis_name="c", subcore_axis_name="s",
                                  num_cores=NC, num_subcores=NS)
    out = pl.kernel(
        _body,
        out_type=jax.ShapeDtypeStruct((B, F, N), jnp.float32),
        mesh=mesh,
        compiler_params=pltpu.CompilerParams(needs_layout_passes=False),
        scratch_types=[
            pltpu.VMEM((N * C,), jnp.float32),
            pltpu.VMEM((N,), jnp.int32),
            pltpu.VMEM((TIMES * F,), jnp.float32),
            pltpu.VMEM((DAYS * F,), jnp.float32),
            [pltpu.VMEM((N,), jnp.float32)] * FQ,
            [pltpu.SemaphoreType.DMA] * FQ,
        ],
    )(x.reshape(B, T, N * C), time_of_day.reshape(-1), day_of_week.reshape(-1))
    return out[..., None]


# speedup vs baseline: 5.4535x; 1.6572x over previous
import jax
import jax.numpy as jnp
from jax import lax
from jax.experimental import pallas as pl
from jax.experimental.pallas import tpu as pltpu
from jax.experimental.pallas import tpu_sc as plsc

B, T, N, C = 64, 12, 10000, 3
TIMES = 288
DAYS = 7
F = 64
NC, NS, L = 2, 16, 16
NW = NC * NS
B_PER_W = B // NW
FQ = 4
TOD_STRIDE = F + 1
DOW_STRIDE = DAYS * F + 1


def _body(x_hbm, tod_hbm, dow_hbm, out_hbm,
          slab_v, pidx_v, todp_v, dowr_v, rows_v, sems):
    wid = lax.axis_index("s") * NC + lax.axis_index("c")
    iota = lax.broadcasted_iota(jnp.int32, (L,), 0)

    pltpu.sync_copy(tod_hbm, slab_v.at[pl.ds(0, TIMES * F)])

    @plsc.parallel_loop(0, TIMES, step=1, unroll=4)
    def _(k):
        for i in range(F // L):
            todp_v[pl.ds(k * TOD_STRIDE + L * i, L)] = (
                slab_v[pl.ds(k * F + L * i, L)])

    pltpu.sync_copy(dow_hbm, slab_v.at[pl.ds(0, DAYS * F)])

    @plsc.parallel_loop(0, L, step=1)
    def _(l):
        for i in range(DAYS * F // L):
            dowr_v[pl.ds(l * DOW_STRIDE + L * i, L)] = (
                slab_v[pl.ds(L * i, L)])

    lane_off = DOW_STRIDE * iota

    for rb in range(B_PER_W):
        b = wid * B_PER_W + rb
        pltpu.sync_copy(x_hbm.at[b, T - 1], slab_v)

        @plsc.parallel_loop(0, N, step=L, unroll=5)
        def _(n0):
            base = 3 * n0 + 3 * iota
            v1 = plsc.load_gather(slab_v, [base + 1])
            v2 = plsc.load_gather(slab_v, [base + 2])
            k65 = (v1 * TIMES).astype(jnp.int32) * TOD_STRIDE
            d64 = v2.astype(jnp.int32) * F
            pidx_v[pl.ds(n0, L)] = k65 | (d64 << 16)

        def quad_body(cc, _):
            f0 = FQ * cc

            @pl.when(cc > 0)
            def _():
                for j in range(FQ):
                    pltpu.make_async_copy(
                        rows_v[j], out_hbm.at[b, 0], sems[j]).wait()

            @plsc.parallel_loop(0, N, step=L, unroll=5)
            def _(n0):
                p = pidx_v[pl.ds(n0, L)]
                k = p & 0xFFFF
                d = lane_off + (p >> 16)
                for j in range(FQ):
                    rows_v[j][pl.ds(n0, L)] = (
                        plsc.load_gather(todp_v, [k + (f0 + j)])
                        + plsc.load_gather(dowr_v, [d + (f0 + j)]))

            for j in range(FQ):
                pltpu.async_copy(rows_v[j], out_hbm.at[b, f0 + j], sems[j])
            return 0

        lax.fori_loop(0, F // FQ, quad_body, 0)
        for j in range(FQ):
            pltpu.make_async_copy(rows_v[j], out_hbm.at[b, 0], sems[j]).wait()


def kernel(x, time_of_day, day_of_week):
    mesh = plsc.VectorSubcoreMesh(core_axis_name="c", subcore_axis_name="s",
                                  num_cores=NC, num_subcores=NS)
    out = pl.kernel(
        _body,
        out_type=jax.ShapeDtypeStruct((B, F, N), jnp.float32),
        mesh=mesh,
        compiler_params=pltpu.CompilerParams(needs_layout_passes=False),
        scratch_types=[
            pltpu.VMEM((N * C,), jnp.float32),
            pltpu.VMEM((N,), jnp.int32),
            pltpu.VMEM((TIMES * TOD_STRIDE,), jnp.float32),
            pltpu.VMEM((L * DOW_STRIDE,), jnp.float32),
            [pltpu.VMEM((N,), jnp.float32)] * FQ,
            [pltpu.SemaphoreType.DMA] * FQ,
        ],
    )(x.reshape(B, T, N * C), time_of_day.reshape(-1), day_of_week.reshape(-1))
    return out[..., None]


# speedup vs baseline: 12.5188x; 2.2955x over previous
import jax
import jax.numpy as jnp
from jax import lax
from jax.experimental import pallas as pl
from jax.experimental.pallas import tpu as pltpu
from jax.experimental.pallas import tpu_sc as plsc

B, T, N, C = 64, 12, 10000, 3
TIMES = 288
DAYS = 7
F = 64
NC, NS, L = 2, 16, 16
NW = NC * NS
B_PER_W = B // NW
FQ = 4
TOD_STRIDE = F + 1
DOW_STRIDE = DAYS * F + 1


def _body(x_hbm, tod_hbm, dow_hbm, out_hbm,
          slab_v, pidx_v, todp_v, dowr_v, rows_v, sems):
    wid = lax.axis_index("s") * NC + lax.axis_index("c")
    iota = lax.broadcasted_iota(jnp.int32, (L,), 0)

    pltpu.sync_copy(tod_hbm, slab_v.at[pl.ds(0, TIMES * F)])

    @plsc.parallel_loop(0, TIMES, step=1, unroll=4)
    def _(k):
        for i in range(F // L):
            todp_v[pl.ds(k * TOD_STRIDE + L * i, L)] = (
                slab_v[pl.ds(k * F + L * i, L)])

    pltpu.sync_copy(dow_hbm, slab_v.at[pl.ds(0, DAYS * F)])

    @plsc.parallel_loop(0, L, step=1)
    def _(l):
        for i in range(DAYS * F // L):
            dowr_v[pl.ds(l * DOW_STRIDE + L * i, L)] = (
                slab_v[pl.ds(L * i, L)])

    lane_off = DOW_STRIDE * iota

    for rb in range(B_PER_W):
        b = wid * B_PER_W + rb
        pltpu.sync_copy(x_hbm.at[b, 0], slab_v)

        @plsc.parallel_loop(0, N, step=L, unroll=5)
        def _(n0):
            base = 3 * n0 + 3 * iota
            v1 = plsc.load_gather(slab_v, [base + 1])
            v2 = plsc.load_gather(slab_v, [base + 2])
            k65 = (v1 * TIMES).astype(jnp.int32) * TOD_STRIDE
            d64 = v2.astype(jnp.int32) * F
            pidx_v[pl.ds(n0, L)] = k65 | (d64 << 16)

        def quad_body(cc, _):
            f0 = FQ * cc

            @pl.when(cc > 0)
            def _():
                for j in range(FQ):
                    pltpu.make_async_copy(
                        rows_v[j], out_hbm.at[b, 0], sems[j]).wait()

            @plsc.parallel_loop(0, N, step=L, unroll=5)
            def _(n0):
                p = pidx_v[pl.ds(n0, L)]
                k = p & 0xFFFF
                d = lane_off + (p >> 16)
                for j in range(FQ):
                    rows_v[j][pl.ds(n0, L)] = (
                        plsc.load_gather(todp_v, [k + (f0 + j)])
                        + plsc.load_gather(dowr_v, [d + (f0 + j)]))

            for j in range(FQ):
                pltpu.async_copy(rows_v[j], out_hbm.at[b, f0 + j], sems[j])
            return 0

        lax.fori_loop(0, F // FQ, quad_body, 0)
        for j in range(FQ):
            pltpu.make_async_copy(rows_v[j], out_hbm.at[b, 0], sems[j]).wait()


def kernel(x, time_of_day, day_of_week):
    mesh = plsc.VectorSubcoreMesh(core_axis_name="c", subcore_axis_name="s",
                                  num_cores=NC, num_subcores=NS)
    out = pl.kernel(
        _body,
        out_type=jax.ShapeDtypeStruct((B, F, N), jnp.float32),
        mesh=mesh,
        compiler_params=pltpu.CompilerParams(needs_layout_passes=False),
        scratch_types=[
            pltpu.VMEM((N * C,), jnp.float32),
            pltpu.VMEM((N,), jnp.int32),
            pltpu.VMEM((TIMES * TOD_STRIDE,), jnp.float32),
            pltpu.VMEM((L * DOW_STRIDE,), jnp.float32),
            [pltpu.VMEM((N,), jnp.float32)] * FQ,
            [pltpu.SemaphoreType.DMA] * FQ,
        ],
    )(x[:, -1].reshape(B, 1, N * C),
      time_of_day.reshape(-1), day_of_week.reshape(-1))
    return out[..., None]


# speedup vs baseline: 13.8361x; 1.1052x over previous
import jax
import jax.numpy as jnp
from jax import lax
from jax.experimental import pallas as pl
from jax.experimental.pallas import tpu as pltpu
from jax.experimental.pallas import tpu_sc as plsc

B, T, N, C = 64, 12, 10000, 3
TIMES = 288
DAYS = 7
F = 64
NC, NS, L = 2, 16, 16
NW = NC * NS
B_PER_W = B // NW
FQ = 4
TOD_STRIDE = F + 1
DOW_STRIDE = DAYS * F + 1


def _body(x_hbm, tod_hbm, dow_hbm, out_hbm,
          slab_v, pidx_v, todp_v, dowr_v, fus_v, rows_v, sems):
    wid = lax.axis_index("s") * NC + lax.axis_index("c")
    iota = lax.broadcasted_iota(jnp.int32, (L,), 0)

    pltpu.sync_copy(tod_hbm, slab_v.at[pl.ds(0, TIMES * F)])

    @plsc.parallel_loop(0, TIMES, step=1, unroll=4)
    def _(k):
        for i in range(F // L):
            todp_v[pl.ds(k * TOD_STRIDE + L * i, L)] = (
                slab_v[pl.ds(k * F + L * i, L)])

    pltpu.sync_copy(dow_hbm, slab_v.at[pl.ds(0, DAYS * F)])

    @plsc.parallel_loop(0, L, step=1)
    def _(l):
        for i in range(DAYS * F // L):
            dowr_v[pl.ds(l * DOW_STRIDE + L * i, L)] = (
                slab_v[pl.ds(L * i, L)])

    lane_off = DOW_STRIDE * iota

    for rb in range(B_PER_W):
        b = wid * B_PER_W + rb
        pltpu.sync_copy(x_hbm.at[b, 0], slab_v)

        def idx_body(i, dminmax):
            n0 = i * L
            base = 3 * n0 + 3 * iota
            v1 = plsc.load_gather(slab_v, [base + 1])
            v2 = plsc.load_gather(slab_v, [base + 2])
            k65 = (v1 * TIMES).astype(jnp.int32) * TOD_STRIDE
            d = v2.astype(jnp.int32)
            pidx_v[pl.ds(n0, L)] = k65 | ((d * F) << 16)
            return (jnp.minimum(dminmax[0], jnp.min(d)),
                    jnp.maximum(dminmax[1], jnp.max(d)))

        dmin, dmax = lax.fori_loop(0, N // L, idx_body,
                                   (jnp.int32(DAYS), jnp.int32(-1)))

        uniform_d = dmin == dmax

        @pl.when(uniform_d)
        def _():
            d0 = dmin * F

            @plsc.parallel_loop(0, TIMES, step=1, unroll=4)
            def _(kk):
                for i in range(F // L):
                    fus_v[pl.ds(kk * TOD_STRIDE + L * i, L)] = (
                        todp_v[pl.ds(kk * TOD_STRIDE + L * i, L)]
                        + dowr_v[pl.ds(d0 + L * i, L)])

        def quad_body(cc, _):
            f0 = FQ * cc

            @pl.when(cc > 0)
            def _():
                for j in range(FQ):
                    pltpu.make_async_copy(
                        rows_v[j], out_hbm.at[b, 0], sems[j]).wait()

            @pl.when(uniform_d)
            def _():
                @plsc.parallel_loop(0, N, step=L, unroll=5)
                def _(n0):
                    k = pidx_v[pl.ds(n0, L)] & 0xFFFF
                    for j in range(FQ):
                        rows_v[j][pl.ds(n0, L)] = (
                            plsc.load_gather(fus_v, [k + (f0 + j)]))

            @pl.when(jnp.logical_not(uniform_d))
            def _():
                @plsc.parallel_loop(0, N, step=L, unroll=5)
                def _(n0):
                    p = pidx_v[pl.ds(n0, L)]
                    k = p & 0xFFFF
                    d = lane_off + (p >> 16)
                    for j in range(FQ):
                        rows_v[j][pl.ds(n0, L)] = (
                            plsc.load_gather(todp_v, [k + (f0 + j)])
                            + plsc.load_gather(dowr_v, [d + (f0 + j)]))

            for j in range(FQ):
                pltpu.async_copy(rows_v[j], out_hbm.at[b, f0 + j], sems[j])
            return 0

        lax.fori_loop(0, F // FQ, quad_body, 0)
        for j in range(FQ):
            pltpu.make_async_copy(rows_v[j], out_hbm.at[b, 0], sems[j]).wait()


def kernel(x, time_of_day, day_of_week):
    mesh = plsc.VectorSubcoreMesh(core_axis_name="c", subcore_axis_name="s",
                                  num_cores=NC, num_subcores=NS)
    out = pl.kernel(
        _body,
        out_type=jax.ShapeDtypeStruct((B, F, N), jnp.float32),
        mesh=mesh,
        compiler_params=pltpu.CompilerParams(needs_layout_passes=False),
        scratch_types=[
            pltpu.VMEM((N * C,), jnp.float32),
            pltpu.VMEM((N,), jnp.int32),
            pltpu.VMEM((TIMES * TOD_STRIDE,), jnp.float32),
            pltpu.VMEM((L * DOW_STRIDE,), jnp.float32),
            pltpu.VMEM((TIMES * TOD_STRIDE,), jnp.float32),
            [pltpu.VMEM((N,), jnp.float32)] * FQ,
            [pltpu.SemaphoreType.DMA] * FQ,
        ],
    )(x[:, -1].reshape(B, 1, N * C),
      time_of_day.reshape(-1), day_of_week.reshape(-1))
    return out[..., None]
